# initial kernel scaffold (unmeasured)
import jax
import jax.numpy as jnp
from jax import lax
from jax.experimental import pallas as pl
from jax.experimental.pallas import tpu as pltpu


def kernel(
    x,
):
    def body(*refs):
        pass

    out_shape = jax.ShapeDtypeStruct(..., jnp.float32)
    return pl.pallas_call(body, out_shape=out_shape)(...)


# baseline (device time: 6950322 ns/iter reference)
import jax
import jax.numpy as jnp
from jax import lax
from jax.experimental import pallas as pl
from jax.experimental.pallas import tpu as pltpu

NDEV = 4
M = 8192
N = 1024
MG = NDEV * M
C = 128
NCHUNK = N // C

_LOG_M = M.bit_length() - 1
_LOG_MG = MG.bit_length() - 1


def _cmpx(v, j, k, flip):
    rows = v.shape[1]
    i = lax.broadcasted_iota(jnp.int32, (1, rows), 1)
    low = (i & j) == 0
    asc = ((i & k) == 0) ^ flip
    p = jnp.where(low, pltpu.roll(v, rows - j, 1), pltpu.roll(v, j, 1))
    take_min = low == asc
    return jnp.where(take_min, jnp.minimum(v, p), jnp.maximum(v, p))


def _stages(v, t_lo, t_hi, flip):

    def stage(t, v):
        k = jnp.int32(1) << (t + 1)

        def one_pass(s, v):
            j = jnp.int32(1) << (t - s)
            return _cmpx(v, j, k, flip)

        return lax.fori_loop(0, t + 1, one_pass, v)

    return lax.fori_loop(t_lo, t_hi, stage, v)


def _body(x_hbm, out_hbm, gbuf, xchunk, gchunk, send_sems, recv_sems,
          load_sem, store_sem):
    my = lax.axis_index("i")
    left = (my + NDEV - 1) % NDEV
    right = (my + 1) % NDEV
    flip = (my % 2) == 1

    barrier = pltpu.get_barrier_semaphore()
    for nbr in (left, right):
        pl.semaphore_signal(barrier, inc=1, device_id=(nbr,),
                            device_id_type=pl.DeviceIdType.MESH)
    pl.semaphore_wait(barrier, 2)

    def local_sort(cc, carry):
        load = pltpu.make_async_copy(x_hbm.at[cc], xchunk, load_sem)
        load.start()
        load.wait()
        xchunk[...] = _stages(xchunk[...], 0, _LOG_M, flip)
        put = pltpu.make_async_copy(
            xchunk, gbuf.at[cc, :, pl.ds(0, M)], store_sem)
        put.start()
        put.wait()
        return carry

    lax.fori_loop(0, NCHUNK, local_sort, 0)

    for h in range(NDEV - 1):
        rdma = pltpu.make_async_remote_copy(
            src_ref=gbuf.at[:, :, pl.ds(h * M, M)],
            dst_ref=gbuf.at[:, :, pl.ds((h + 1) * M, M)],
            send_sem=send_sems.at[h],
            recv_sem=recv_sems.at[h],
            device_id=(right,),
            device_id_type=pl.DeviceIdType.MESH,
        )
        rdma.start()
        rdma.wait()

    def merge(cc, carry):
        load = pltpu.make_async_copy(gbuf.at[cc], gchunk, load_sem)
        load.start()
        load.wait()
        gchunk[...] = _stages(gchunk[...], _LOG_M, _LOG_MG, False)
        store = pltpu.make_async_copy(
            gchunk.at[:, pl.ds(my * M, M)], out_hbm.at[cc], store_sem)
        store.start()
        store.wait()
        return carry

    lax.fori_loop(0, NCHUNK, merge, 0)


def kernel(x):
    xt = x.astype(jnp.bfloat16).T.reshape(NCHUNK, C, M)
    out, _ = pl.pallas_call(
        _body,
        out_shape=(
            jax.ShapeDtypeStruct((NCHUNK, C, M), jnp.bfloat16),
            jax.ShapeDtypeStruct((NCHUNK, C, MG), jnp.bfloat16),
        ),
        in_specs=[pl.BlockSpec(memory_space=pltpu.MemorySpace.HBM)],
        out_specs=(
            pl.BlockSpec(memory_space=pltpu.MemorySpace.HBM),
            pl.BlockSpec(memory_space=pltpu.MemorySpace.HBM),
        ),
        scratch_shapes=[
            pltpu.VMEM((C, M), jnp.bfloat16),
            pltpu.VMEM((C, MG), jnp.bfloat16),
            pltpu.SemaphoreType.DMA((NDEV - 1,)),
            pltpu.SemaphoreType.DMA((NDEV - 1,)),
            pltpu.SemaphoreType.DMA,
            pltpu.SemaphoreType.DMA,
        ],
        compiler_params=pltpu.CompilerParams(
            collective_id=0, vmem_limit_bytes=56 * 1024 * 1024),
    )(xt)
    return out.reshape(N, M).T
